# baseline (speedup 1.0000x reference)
import functools

import jax
import jax.numpy as jnp
from jax import lax
from jax.experimental import pallas as pl
from jax.experimental.pallas import tpu as pltpu
from jax.experimental.pallas import tpu_sc as plsc

D_MODEL = 1024
MAX_LEN = 128
VISIBLE_RANGE = 9.0
NSIDE = 10
NROWS = 104
NSIDX = 112

NC, NS, L = 2, 16, 16
NW = NC * NS

B_TOTAL = 4 * 2048
B_PER_W = B_TOTAL // NW
CHUNK = 8
N_PAIR = B_PER_W // (2 * CHUNK)

_MAGIC = 2.0**23


def _round_half_even(v):
    return ((v + _MAGIC) - _MAGIC).astype(jnp.int32)


def _sc_gather(tokens_flat, enc_flat):
    mesh = plsc.VectorSubcoreMesh(core_axis_name="c", subcore_axis_name="s")

    @functools.partial(
        pl.kernel,
        mesh=mesh,
        out_type=jax.ShapeDtypeStruct((B_TOTAL, D_MODEL), jnp.float32),
        scratch_types=[
            pltpu.VMEM((B_PER_W * 2,), jnp.float32),
            pltpu.VMEM((B_PER_W,), jnp.int32),
            pltpu.VMEM((NSIDX,), jnp.int32),
            pltpu.VMEM((NROWS, D_MODEL), jnp.float32),
            pltpu.VMEM((CHUNK, D_MODEL), jnp.float32),
            pltpu.VMEM((CHUNK, D_MODEL), jnp.float32),
            pltpu.SemaphoreType.DMA,
            pltpu.SemaphoreType.DMA,
            pltpu.SemaphoreType.DMA,
        ],
    )
    def k(tok_hbm, enc_hbm, out_hbm, tok_v, idx_v, sidx_v, table_v,
          buf0, buf1, tsem, w0, w1):
        wid = lax.axis_index("s") * NC + lax.axis_index("c")
        base = wid * B_PER_W

        lanes = lax.iota(jnp.int32, L)
        for c in range(6):
            kk = lanes + (c * L)
            sidx_v[pl.ds(c * L, L)] = (
                lax.div(kk, NSIDE) * MAX_LEN + lax.rem(kk, NSIDE))
        kk = jnp.minimum(lanes + 88, NSIDE * NSIDE - 1)
        sidx_v[pl.ds(6 * L, L)] = (
            lax.div(kk, NSIDE) * MAX_LEN + lax.rem(kk, NSIDE))
        pltpu.async_copy(
            enc_hbm.at[sidx_v.at[pl.ds(0, 6 * L)]],
            table_v.at[pl.ds(0, 6 * L)], tsem)
        table_cp = pltpu.async_copy(
            enc_hbm.at[sidx_v.at[pl.ds(6 * L, L)]],
            table_v.at[pl.ds(88, L)], tsem)

        pltpu.sync_copy(tok_hbm.at[pl.ds(base, B_PER_W)],
                        tok_v.at[pl.ds(0, B_PER_W)])
        pltpu.sync_copy(tok_hbm.at[pl.ds(B_TOTAL + base, B_PER_W)],
                        tok_v.at[pl.ds(B_PER_W, B_PER_W)])

        for i in range(B_PER_W // L):
            x = tok_v[pl.ds(i * L, L)]
            y = tok_v[pl.ds(B_PER_W + i * L, L)]
            rx = _round_half_even(x * VISIBLE_RANGE)
            ry = _round_half_even(y * VISIBLE_RANGE)
            idx_v[pl.ds(i * L, L)] = rx * NSIDE + ry

        pltpu.make_async_copy(
            enc_hbm.at[sidx_v.at[pl.ds(0, 6 * L)]],
            table_v.at[pl.ds(0, 6 * L)], tsem).wait()
        table_cp.wait()

        del table_cp

        def fill(buf, rows_vec, lo):
            for t in range(CHUNK):
                row = rows_vec[lo + t]
                for c in range(D_MODEL // L):
                    buf[t, pl.ds(c * L, L)] = table_v[row, pl.ds(c * L, L)]

        def body(p, _):
            rows_vec = idx_v[pl.ds(p * (2 * CHUNK), 2 * CHUNK)]
            j0 = p * 2

            @pl.when(p > 0)
            def _wait_prev():
                pltpu.make_async_copy(
                    buf0, out_hbm.at[pl.ds(base + (j0 - 2) * CHUNK, CHUNK)],
                    w0).wait()

            fill(buf0, rows_vec, 0)
            pltpu.async_copy(
                buf0, out_hbm.at[pl.ds(base + j0 * CHUNK, CHUNK)], w0)

            @pl.when(p > 0)
            def _wait_prev1():
                pltpu.make_async_copy(
                    buf1, out_hbm.at[pl.ds(base + (j0 - 1) * CHUNK, CHUNK)],
                    w1).wait()

            fill(buf1, rows_vec, CHUNK)
            pltpu.async_copy(
                buf1, out_hbm.at[pl.ds(base + (j0 + 1) * CHUNK, CHUNK)], w1)
            return _

        lax.fori_loop(0, N_PAIR, body, None)
        last = 2 * (N_PAIR - 1)
        pltpu.make_async_copy(
            buf0, out_hbm.at[pl.ds(base + last * CHUNK, CHUNK)], w0).wait()
        pltpu.make_async_copy(
            buf1, out_hbm.at[pl.ds(base + (last + 1) * CHUNK, CHUNK)],
            w1).wait()

    return k(tokens_flat, enc_flat)


def kernel(tokens, encoding):
    b, s, _ = tokens.shape
    tokens_flat = tokens.reshape(b * s, 2).T.reshape(b * s * 2)
    enc_flat = encoding.reshape(MAX_LEN * MAX_LEN, D_MODEL)
    out = _sc_gather(tokens_flat, enc_flat)
    return out.reshape(b, s, D_MODEL)


# speedup vs baseline: 1.6063x; 1.6063x over previous
import functools

import jax
import jax.numpy as jnp
from jax import lax
from jax.experimental import pallas as pl
from jax.experimental.pallas import tpu as pltpu
from jax.experimental.pallas import tpu_sc as plsc

D_MODEL = 1024
DH = D_MODEL // 2
MAX_LEN = 128
VISIBLE_RANGE = 9.0
NSIDE = 10
TROWS = 16

NC, NS, L = 2, 16, 16
NW = NC * NS

B_TOTAL = 4 * 2048
B_PER_W = B_TOTAL // NW
CHUNK = 16
N_PAIR = B_PER_W // (2 * CHUNK)

_MAGIC = 2.0**23


def _round_half_even(v):
    return ((v + _MAGIC) - _MAGIC).astype(jnp.int32)


def _sc_gather(tokens_flat, enc_flat):
    mesh = plsc.VectorSubcoreMesh(core_axis_name="c", subcore_axis_name="s")

    @functools.partial(
        pl.kernel,
        mesh=mesh,
        out_type=jax.ShapeDtypeStruct((B_TOTAL, D_MODEL), jnp.float32),
        scratch_types=[
            pltpu.VMEM((B_PER_W * 2,), jnp.float32),
            pltpu.VMEM((B_PER_W * 2,), jnp.int32),
            pltpu.VMEM((2 * TROWS,), jnp.int32),
            pltpu.VMEM((TROWS, D_MODEL), jnp.float32),
            pltpu.VMEM((TROWS, D_MODEL), jnp.float32),
            pltpu.VMEM((CHUNK, D_MODEL), jnp.float32),
            pltpu.VMEM((CHUNK, D_MODEL), jnp.float32),
            pltpu.SemaphoreType.DMA,
            pltpu.SemaphoreType.DMA,
            pltpu.SemaphoreType.DMA,
        ],
    )
    def k(tok_hbm, enc_hbm, out_hbm, tok_v, idx_v, sidx_v, xtab, ytab,
          buf0, buf1, tsem, w0, w1):
        wid = lax.axis_index("s") * NC + lax.axis_index("c")
        base = wid * B_PER_W

        lanes = lax.iota(jnp.int32, L)
        rows10 = jnp.minimum(lanes, NSIDE - 1)
        sidx_v[pl.ds(0, L)] = rows10 * MAX_LEN
        sidx_v[pl.ds(L, L)] = rows10
        pltpu.async_copy(enc_hbm.at[sidx_v.at[pl.ds(0, L)]], xtab, tsem)
        pltpu.async_copy(enc_hbm.at[sidx_v.at[pl.ds(L, L)]], ytab, tsem)

        pltpu.sync_copy(tok_hbm.at[pl.ds(base, B_PER_W)],
                        tok_v.at[pl.ds(0, B_PER_W)])
        pltpu.sync_copy(tok_hbm.at[pl.ds(B_TOTAL + base, B_PER_W)],
                        tok_v.at[pl.ds(B_PER_W, B_PER_W)])

        for i in range(2 * B_PER_W // L):
            idx_v[pl.ds(i * L, L)] = _round_half_even(
                tok_v[pl.ds(i * L, L)] * VISIBLE_RANGE)

        pltpu.make_async_copy(
            enc_hbm.at[sidx_v.at[pl.ds(0, L)]], xtab, tsem).wait()
        pltpu.make_async_copy(
            enc_hbm.at[sidx_v.at[pl.ds(L, L)]], ytab, tsem).wait()

        def fill(buf, rxs, rys):
            for t in range(CHUNK):
                rowx = rxs[t]
                rowy = rys[t]

                @plsc.parallel_loop(0, DH // L, unroll=8)
                def _copy_x(c):
                    buf[t, pl.ds(c * L, L)] = xtab[rowx, pl.ds(c * L, L)]

                @plsc.parallel_loop(0, DH // L, unroll=8)
                def _copy_y(c):
                    buf[t, pl.ds(DH + c * L, L)] = (
                        ytab[rowy, pl.ds(DH + c * L, L)])

        def body(p, _):
            rxs0 = idx_v[pl.ds(p * (2 * CHUNK), CHUNK)]
            rxs1 = idx_v[pl.ds(p * (2 * CHUNK) + CHUNK, CHUNK)]
            rys0 = idx_v[pl.ds(B_PER_W + p * (2 * CHUNK), CHUNK)]
            rys1 = idx_v[pl.ds(B_PER_W + p * (2 * CHUNK) + CHUNK, CHUNK)]
            j0 = p * 2

            @pl.when(p > 0)
            def _wait_prev0():
                pltpu.make_async_copy(
                    buf0, out_hbm.at[pl.ds(base + (j0 - 2) * CHUNK, CHUNK)],
                    w0).wait()

            fill(buf0, rxs0, rys0)
            pltpu.async_copy(
                buf0, out_hbm.at[pl.ds(base + j0 * CHUNK, CHUNK)], w0)

            @pl.when(p > 0)
            def _wait_prev1():
                pltpu.make_async_copy(
                    buf1, out_hbm.at[pl.ds(base + (j0 - 1) * CHUNK, CHUNK)],
                    w1).wait()

            fill(buf1, rxs1, rys1)
            pltpu.async_copy(
                buf1, out_hbm.at[pl.ds(base + (j0 + 1) * CHUNK, CHUNK)], w1)
            return _

        lax.fori_loop(0, N_PAIR, body, None)
        last = 2 * (N_PAIR - 1)
        pltpu.make_async_copy(
            buf0, out_hbm.at[pl.ds(base + last * CHUNK, CHUNK)], w0).wait()
        pltpu.make_async_copy(
            buf1, out_hbm.at[pl.ds(base + (last + 1) * CHUNK, CHUNK)],
            w1).wait()

    return k(tokens_flat, enc_flat)


def kernel(tokens, encoding):
    b, s, _ = tokens.shape
    tokens_flat = tokens.reshape(b * s, 2).T.reshape(b * s * 2)
    enc_flat = encoding.reshape(MAX_LEN * MAX_LEN, D_MODEL)
    out = _sc_gather(tokens_flat, enc_flat)
    return out.reshape(b, s, D_MODEL)


# speedup vs baseline: 2.2507x; 1.4011x over previous
import functools

import jax
import jax.numpy as jnp
from jax import lax
from jax.experimental import pallas as pl
from jax.experimental.pallas import tpu as pltpu
from jax.experimental.pallas import tpu_sc as plsc

D_MODEL = 1024
MAX_LEN = 128
VISIBLE_RANGE = 9.0
NSIDE = 10
NROWS = 104
NSIDX = 112

NC, NS, L = 2, 16, 16
NW = NC * NS

B_TOTAL = 4 * 2048
B_PER_W = B_TOTAL // NW
CHUNK = 8
N_PAIR = B_PER_W // (2 * CHUNK)

_MAGIC = 2.0**23


def _round_half_even(v):
    return ((v + _MAGIC) - _MAGIC).astype(jnp.int32)


def _sc_gather(tokens_flat, enc_flat):
    mesh = plsc.VectorSubcoreMesh(core_axis_name="c", subcore_axis_name="s")

    @functools.partial(
        pl.kernel,
        mesh=mesh,
        out_type=jax.ShapeDtypeStruct((B_TOTAL, D_MODEL), jnp.float32),
        scratch_types=[
            pltpu.VMEM((B_PER_W * 2,), jnp.float32),
            pltpu.VMEM((B_PER_W,), jnp.int32),
            pltpu.VMEM((NSIDX,), jnp.int32),
            pltpu.VMEM((NROWS, D_MODEL), jnp.float32),
            pltpu.VMEM((CHUNK, D_MODEL), jnp.float32),
            pltpu.VMEM((CHUNK, D_MODEL), jnp.float32),
            pltpu.VMEM_SHARED((NROWS, D_MODEL), jnp.float32),
            pltpu.SemaphoreType.DMA,
            pltpu.SemaphoreType.DMA,
            pltpu.SemaphoreType.DMA,
        ],
    )
    def k(tok_hbm, enc_hbm, out_hbm, tok_v, idx_v, sidx_v, table_v,
          buf0, buf1, table_sh, tsem, w0, w1):
        sid = lax.axis_index("s")
        wid = sid * NC + lax.axis_index("c")
        base = wid * B_PER_W

        @pl.when(sid == 0)
        def _stage_gather():
            lanes = lax.iota(jnp.int32, L)
            for c in range(6):
                kk = lanes + (c * L)
                sidx_v[pl.ds(c * L, L)] = (
                    lax.div(kk, NSIDE) * MAX_LEN + lax.rem(kk, NSIDE))
            kk = jnp.minimum(lanes + 88, NSIDE * NSIDE - 1)
            sidx_v[pl.ds(6 * L, L)] = (
                lax.div(kk, NSIDE) * MAX_LEN + lax.rem(kk, NSIDE))
            pltpu.async_copy(
                enc_hbm.at[sidx_v.at[pl.ds(0, 6 * L)]],
                table_v.at[pl.ds(0, 6 * L)], tsem)
            pltpu.async_copy(
                enc_hbm.at[sidx_v.at[pl.ds(6 * L, L)]],
                table_v.at[pl.ds(88, L)], tsem)

        pltpu.sync_copy(tok_hbm.at[pl.ds(base, B_PER_W)],
                        tok_v.at[pl.ds(0, B_PER_W)])
        pltpu.sync_copy(tok_hbm.at[pl.ds(B_TOTAL + base, B_PER_W)],
                        tok_v.at[pl.ds(B_PER_W, B_PER_W)])

        for i in range(B_PER_W // L):
            x = tok_v[pl.ds(i * L, L)]
            y = tok_v[pl.ds(B_PER_W + i * L, L)]
            rx = _round_half_even(x * VISIBLE_RANGE)
            ry = _round_half_even(y * VISIBLE_RANGE)
            idx_v[pl.ds(i * L, L)] = rx * NSIDE + ry

        @pl.when(sid == 0)
        def _stage_publish():
            pltpu.make_async_copy(
                enc_hbm.at[sidx_v.at[pl.ds(0, 6 * L)]],
                table_v.at[pl.ds(0, 6 * L)], tsem).wait()
            pltpu.make_async_copy(
                enc_hbm.at[sidx_v.at[pl.ds(6 * L, L)]],
                table_v.at[pl.ds(88, L)], tsem).wait()
            pltpu.sync_copy(table_v, table_sh)

        plsc.subcore_barrier()

        @pl.when(sid != 0)
        def _stage_pull():
            pltpu.sync_copy(table_sh, table_v)

        def fill(buf, rows_vec, lo):
            for t in range(CHUNK):
                row = rows_vec[lo + t]

                @plsc.parallel_loop(0, D_MODEL // L, unroll=8)
                def _copy_row(c):
                    buf[t, pl.ds(c * L, L)] = table_v[row, pl.ds(c * L, L)]

        def body(p, _):
            rows_vec = idx_v[pl.ds(p * (2 * CHUNK), 2 * CHUNK)]
            j0 = p * 2

            @pl.when(p > 0)
            def _wait_prev():
                pltpu.make_async_copy(
                    buf0, out_hbm.at[pl.ds(base + (j0 - 2) * CHUNK, CHUNK)],
                    w0).wait()

            fill(buf0, rows_vec, 0)
            pltpu.async_copy(
                buf0, out_hbm.at[pl.ds(base + j0 * CHUNK, CHUNK)], w0)

            @pl.when(p > 0)
            def _wait_prev1():
                pltpu.make_async_copy(
                    buf1, out_hbm.at[pl.ds(base + (j0 - 1) * CHUNK, CHUNK)],
                    w1).wait()

            fill(buf1, rows_vec, CHUNK)
            pltpu.async_copy(
                buf1, out_hbm.at[pl.ds(base + (j0 + 1) * CHUNK, CHUNK)], w1)
            return _

        lax.fori_loop(0, N_PAIR, body, None)
        last = 2 * (N_PAIR - 1)
        pltpu.make_async_copy(
            buf0, out_hbm.at[pl.ds(base + last * CHUNK, CHUNK)], w0).wait()
        pltpu.make_async_copy(
            buf1, out_hbm.at[pl.ds(base + (last + 1) * CHUNK, CHUNK)],
            w1).wait()

    return k(tokens_flat, enc_flat)


def kernel(tokens, encoding):
    b, s, _ = tokens.shape
    tokens_flat = tokens.reshape(b * s, 2).T.reshape(b * s * 2)
    enc_flat = encoding.reshape(MAX_LEN * MAX_LEN, D_MODEL)
    out = _sc_gather(tokens_flat, enc_flat)
    return out.reshape(b, s, D_MODEL)
